# baseline (speedup 1.0000x reference)
import math

import jax
import jax.numpy as jnp
from jax.experimental import pallas as pl

NB = 9
POOL = 8
H = W = 384
HP = H // POOL
WP = W // POOL
CPB = 2
NBLK = (HP // CPB) * (WP // CPB)


def _channel_hist(img, pv, pvt):
    img = img.astype(jnp.bfloat16).astype(jnp.float32)

    t = jnp.dot(pv, img, preferred_element_type=jnp.float32)
    s = jnp.dot(img, pvt, preferred_element_type=jnp.float32)
    tl = jnp.concatenate([t[:, 1:2], t[:, 0:W - 1]], axis=1)
    tr = jnp.concatenate([t[:, 1:W], t[:, W - 2:W - 1]], axis=1)
    gx = tl - tr
    su = jnp.concatenate([s[1:2, :], s[0:H - 1, :]], axis=0)
    sd = jnp.concatenate([s[1:H, :], s[H - 2:H - 1, :]], axis=0)
    gy = su - sd

    norm = jnp.sqrt(gx * gx + gy * gy)

    flip = (gx < 0.0) | ((gx == 0.0) & (gy < 0.0))
    sgn = jnp.where(flip, -1.0, 1.0)
    a = gx * sgn
    b = gy * sgn

    vals = [norm]
    for k in range(1, NB):
        al = k * math.pi / NB
        ind = a * (math.cos(al) / math.sin(al)) >= b
        vals.append(jnp.where(ind, norm, 0.0))

    ri = jax.lax.broadcasted_iota(jnp.int32, (HP, H), 0)
    rj = jax.lax.broadcasted_iota(jnp.int32, (HP, H), 1)
    prt = (rj // POOL == ri).astype(jnp.float32)
    ji = jax.lax.broadcasted_iota(jnp.int32, (W, WP), 0)
    jo = jax.lax.broadcasted_iota(jnp.int32, (W, WP), 1)
    pmat = (ji // POOL == jo).astype(jnp.float32)
    pooled = [
        jnp.dot(jnp.dot(prt, v, preferred_element_type=jnp.float32), pmat,
                preferred_element_type=jnp.float32)
        for v in vals
    ]

    hs = [pooled[k] - pooled[k + 1] if k < NB - 1 else pooled[k]
          for k in range(NB)]

    ssq = hs[0] * hs[0]
    for k in range(1, NB):
        ssq = ssq + hs[k] * hs[k]
    inv = 1.0 / jnp.maximum(jnp.sqrt(ssq), 1e-12)
    return [h * inv for h in hs]


def _hog_kernel(x_ref, o_ref):
    di = jax.lax.broadcasted_iota(jnp.int32, (H, H), 0)
    dj = jax.lax.broadcasted_iota(jnp.int32, (H, H), 1)
    ad = jnp.abs(di - dj)
    pv = jnp.where(ad <= 1, (2 - ad).astype(jnp.float32), 0.0)
    edge = ((di == 0) & (dj == 1)) | ((di == H - 1) & (dj == H - 2))
    pv = pv + edge.astype(jnp.float32)
    pvt = pv.T

    planes = []
    for c in range(3):
        planes.extend(_channel_hist(x_ref[0, c], pv, pvt))
    hsn = jnp.stack(planes, axis=0)

    nbh = HP // CPB
    nbw = WP // CPB
    qtot = 3 * NB * CPB * CPB

    ci = jax.lax.broadcasted_iota(jnp.int32, (WP, nbw), 0)
    bi = jax.lax.broadcasted_iota(jnp.int32, (WP, nbw), 1)
    csel0 = (ci == 2 * bi).astype(jnp.float32)
    csel1 = (ci == 2 * bi + 1).astype(jnp.float32)

    si = jax.lax.broadcasted_iota(jnp.int32, (qtot, qtot), 0)
    li = jax.lax.broadcasted_iota(jnp.int32, (qtot, qtot), 1)
    pw_s = si // 54
    rem = si % 54
    tgt = (rem // 2) * 4 + (rem % 2) * 2 + pw_s
    perm = (li == tgt).astype(jnp.float32)

    for bh in range(nbh):
        q = hsn[:, 2 * bh:2 * bh + 2, :].reshape(54, WP)
        sub0 = jnp.dot(q, csel0, preferred_element_type=jnp.float32)
        sub1 = jnp.dot(q, csel1, preferred_element_type=jnp.float32)
        scat = jnp.concatenate([sub0, sub1], axis=0)
        tbh = jnp.dot(scat.T, perm, preferred_element_type=jnp.float32)
        o_ref[0, bh * nbw:(bh + 1) * nbw, :] = tbh


def kernel(x, weight_x, weight_y):
    del weight_x, weight_y
    bsz, c = x.shape[0], x.shape[1]
    qq = c * NB * CPB * CPB
    return pl.pallas_call(
        _hog_kernel,
        grid=(bsz,),
        in_specs=[pl.BlockSpec((1, c, H, W), lambda i: (i, 0, 0, 0))],
        out_specs=pl.BlockSpec((1, NBLK, qq), lambda i: (i, 0, 0)),
        out_shape=jax.ShapeDtypeStruct((bsz, NBLK, qq), jnp.float32),
    )(x)


# speedup vs baseline: 1.0861x; 1.0861x over previous
import math

import jax
import jax.numpy as jnp
from jax.experimental import pallas as pl

NB = 9
POOL = 8
H = W = 384
HP = H // POOL
WP = W // POOL
CPB = 2
NBLK = (HP // CPB) * (WP // CPB)


def _channel_hist(img):
    img = img.astype(jnp.bfloat16).astype(jnp.float32)

    xp = jnp.concatenate([img[1:2, :], img, img[H - 2:H - 1, :]], axis=0)
    t = xp[0:H, :] + 2.0 * xp[1:H + 1, :] + xp[2:H + 2, :]
    sl = jnp.concatenate([img[:, 1:2], img[:, 0:W - 1]], axis=1)
    sr = jnp.concatenate([img[:, 1:W], img[:, W - 2:W - 1]], axis=1)
    s = sl + 2.0 * img + sr
    tl = jnp.concatenate([t[:, 1:2], t[:, 0:W - 1]], axis=1)
    tr = jnp.concatenate([t[:, 1:W], t[:, W - 2:W - 1]], axis=1)
    gx = tl - tr
    su = jnp.concatenate([s[1:2, :], s[0:H - 1, :]], axis=0)
    sd = jnp.concatenate([s[1:H, :], s[H - 2:H - 1, :]], axis=0)
    gy = su - sd

    norm = jnp.sqrt(gx * gx + gy * gy)

    flip = (gx < 0.0) | ((gx == 0.0) & (gy < 0.0))
    sgn = jnp.where(flip, -1.0, 1.0)
    a = gx * sgn
    b = gy * sgn

    vals = [norm]
    for k in range(1, NB):
        al = k * math.pi / NB
        ind = a * (math.cos(al) / math.sin(al)) >= b
        vals.append(jnp.where(ind, norm, 0.0))

    ri = jax.lax.broadcasted_iota(jnp.int32, (HP, H), 0)
    rj = jax.lax.broadcasted_iota(jnp.int32, (HP, H), 1)
    prt = (rj // POOL == ri).astype(jnp.float32)
    ji = jax.lax.broadcasted_iota(jnp.int32, (W, WP), 0)
    jo = jax.lax.broadcasted_iota(jnp.int32, (W, WP), 1)
    pmat = (ji // POOL == jo).astype(jnp.float32)
    pooled = [
        jnp.dot(jnp.dot(prt, v, preferred_element_type=jnp.float32), pmat,
                preferred_element_type=jnp.float32)
        for v in vals
    ]

    hs = [pooled[k] - pooled[k + 1] if k < NB - 1 else pooled[k]
          for k in range(NB)]

    ssq = hs[0] * hs[0]
    for k in range(1, NB):
        ssq = ssq + hs[k] * hs[k]
    inv = 1.0 / jnp.maximum(jnp.sqrt(ssq), 1e-12)
    return [h * inv for h in hs]


def _hog_kernel(x_ref, o_ref):
    planes = []
    for c in range(3):
        planes.extend(_channel_hist(x_ref[0, c]))
    hsn = jnp.stack(planes, axis=0)

    nbh = HP // CPB
    nbw = WP // CPB
    qtot = 3 * NB * CPB * CPB

    ci = jax.lax.broadcasted_iota(jnp.int32, (WP, nbw), 0)
    bi = jax.lax.broadcasted_iota(jnp.int32, (WP, nbw), 1)
    csel0 = (ci == 2 * bi).astype(jnp.float32)
    csel1 = (ci == 2 * bi + 1).astype(jnp.float32)

    si = jax.lax.broadcasted_iota(jnp.int32, (qtot, qtot), 0)
    li = jax.lax.broadcasted_iota(jnp.int32, (qtot, qtot), 1)
    pw_s = si // 54
    rem = si % 54
    tgt = (rem // 2) * 4 + (rem % 2) * 2 + pw_s
    perm = (li == tgt).astype(jnp.float32)

    for bh in range(nbh):
        q = hsn[:, 2 * bh:2 * bh + 2, :].reshape(54, WP)
        sub0 = jnp.dot(q, csel0, preferred_element_type=jnp.float32)
        sub1 = jnp.dot(q, csel1, preferred_element_type=jnp.float32)
        scat = jnp.concatenate([sub0, sub1], axis=0)
        tbh = jnp.dot(scat.T, perm, preferred_element_type=jnp.float32)
        o_ref[0, bh * nbw:(bh + 1) * nbw, :] = tbh


def kernel(x, weight_x, weight_y):
    del weight_x, weight_y
    bsz, c = x.shape[0], x.shape[1]
    qq = c * NB * CPB * CPB
    return pl.pallas_call(
        _hog_kernel,
        grid=(bsz,),
        in_specs=[pl.BlockSpec((1, c, H, W), lambda i: (i, 0, 0, 0))],
        out_specs=pl.BlockSpec((1, NBLK, qq), lambda i: (i, 0, 0)),
        out_shape=jax.ShapeDtypeStruct((bsz, NBLK, qq), jnp.float32),
    )(x)


# speedup vs baseline: 1.6461x; 1.5157x over previous
import math

import jax
import jax.numpy as jnp
from jax.experimental import pallas as pl

NB = 9
POOL = 8
H = W = 384
HP = H // POOL
WP = W // POOL
CPB = 2
NBLK = (HP // CPB) * (WP // CPB)


def _channel_hist(img):
    img = img.astype(jnp.bfloat16).astype(jnp.float32)

    xp = jnp.concatenate([img[1:2, :], img, img[H - 2:H - 1, :]], axis=0)
    t = xp[0:H, :] + 2.0 * xp[1:H + 1, :] + xp[2:H + 2, :]
    sl = jnp.concatenate([img[:, 1:2], img[:, 0:W - 1]], axis=1)
    sr = jnp.concatenate([img[:, 1:W], img[:, W - 2:W - 1]], axis=1)
    s = sl + 2.0 * img + sr
    tl = jnp.concatenate([t[:, 1:2], t[:, 0:W - 1]], axis=1)
    tr = jnp.concatenate([t[:, 1:W], t[:, W - 2:W - 1]], axis=1)
    gx = tl - tr
    su = jnp.concatenate([s[1:2, :], s[0:H - 1, :]], axis=0)
    sd = jnp.concatenate([s[1:H, :], s[H - 2:H - 1, :]], axis=0)
    gy = su - sd

    norm = jnp.sqrt(gx * gx + gy * gy)

    flip = (gx < 0.0) | ((gx == 0.0) & (gy < 0.0))
    sgn = jnp.where(flip, -1.0, 1.0)
    a = gx * sgn
    b = gy * sgn

    vals = [norm]
    for k in range(1, NB):
        al = k * math.pi / NB
        ind = a * (math.cos(al) / math.sin(al)) >= b
        vals.append(jnp.where(ind, norm, 0.0))

    ri = jax.lax.broadcasted_iota(jnp.int32, (HP, H), 0)
    rj = jax.lax.broadcasted_iota(jnp.int32, (HP, H), 1)
    prt = (rj // POOL == ri).astype(jnp.float32)
    ji = jax.lax.broadcasted_iota(jnp.int32, (W, WP), 0)
    jo = jax.lax.broadcasted_iota(jnp.int32, (W, WP), 1)
    pmat = (ji // POOL == 2 * (jo % 24) + jo // 24).astype(jnp.float32)
    pooled = [
        jnp.dot(jnp.dot(prt, v, preferred_element_type=jnp.float32), pmat,
                preferred_element_type=jnp.float32)
        for v in vals
    ]

    hs = [pooled[k] - pooled[k + 1] if k < NB - 1 else pooled[k]
          for k in range(NB)]

    ssq = hs[0] * hs[0]
    for k in range(1, NB):
        ssq = ssq + hs[k] * hs[k]
    inv = 1.0 / jnp.maximum(jnp.sqrt(ssq), 1e-12)
    return [h * inv for h in hs]


def _hog_kernel(x_ref, o_ref):
    planes = []
    for c in range(3):
        planes.extend(_channel_hist(x_ref[0, c]))
    hsn = jnp.stack(planes, axis=0)

    nbh = HP // CPB
    nbw = WP // CPB
    qtot = 3 * NB * CPB * CPB

    si = jax.lax.broadcasted_iota(jnp.int32, (54, qtot), 0)
    li = jax.lax.broadcasted_iota(jnp.int32, (54, qtot), 1)
    e0 = (li == 2 * si).astype(jnp.float32)
    e1 = (li == 2 * si + 1).astype(jnp.float32)

    for bh in range(nbh):
        q = hsn[:, 2 * bh:2 * bh + 2, :].reshape(54, WP)
        tq = q.T
        r0 = tq[0:nbw]
        r1 = tq[nbw:2 * nbw]
        tbh = (jnp.dot(r0, e0, preferred_element_type=jnp.float32)
               + jnp.dot(r1, e1, preferred_element_type=jnp.float32))
        o_ref[0, bh * nbw:(bh + 1) * nbw, :] = tbh


def kernel(x, weight_x, weight_y):
    del weight_x, weight_y
    bsz, c = x.shape[0], x.shape[1]
    qq = c * NB * CPB * CPB
    return pl.pallas_call(
        _hog_kernel,
        grid=(bsz,),
        in_specs=[pl.BlockSpec((1, c, H, W), lambda i: (i, 0, 0, 0))],
        out_specs=pl.BlockSpec((1, NBLK, qq), lambda i: (i, 0, 0)),
        out_shape=jax.ShapeDtypeStruct((bsz, NBLK, qq), jnp.float32),
    )(x)


# speedup vs baseline: 1.7669x; 1.0734x over previous
import math

import jax
import jax.numpy as jnp
from jax.experimental import pallas as pl

NB = 9
POOL = 8
H = W = 384
HP = H // POOL
WP = W // POOL
CPB = 2
NBLK = (HP // CPB) * (WP // CPB)


def _channel_hist(img):
    img = img.astype(jnp.bfloat16).astype(jnp.float32)

    img_um = jnp.concatenate([img[1:2, :], img[0:H - 1, :]], axis=0)
    img_dp = jnp.concatenate([img[1:H, :], img[H - 2:H - 1, :]], axis=0)
    t = img_um + 2.0 * img + img_dp
    sl = jnp.concatenate([img[:, 1:2], img[:, 0:W - 1]], axis=1)
    sr = jnp.concatenate([img[:, 1:W], img[:, W - 2:W - 1]], axis=1)
    s = sl + 2.0 * img + sr
    tl = jnp.concatenate([t[:, 1:2], t[:, 0:W - 1]], axis=1)
    tr = jnp.concatenate([t[:, 1:W], t[:, W - 2:W - 1]], axis=1)
    gx = tl - tr
    su = jnp.concatenate([s[1:2, :], s[0:H - 1, :]], axis=0)
    sd = jnp.concatenate([s[1:H, :], s[H - 2:H - 1, :]], axis=0)
    gy = su - sd

    norm = jnp.sqrt(gx * gx + gy * gy)

    r = gy / gx
    r = jnp.where((gx == 0.0) & (gy < 0.0), jnp.inf, r)

    vals = [norm]
    for k in range(1, NB):
        al = k * math.pi / NB
        ind = r <= (math.cos(al) / math.sin(al))
        vals.append(jnp.where(ind, norm, 0.0))

    ri = jax.lax.broadcasted_iota(jnp.int32, (HP, H), 0)
    rj = jax.lax.broadcasted_iota(jnp.int32, (HP, H), 1)
    prt = (rj // POOL == ri).astype(jnp.float32)
    ji = jax.lax.broadcasted_iota(jnp.int32, (W, WP), 0)
    jo = jax.lax.broadcasted_iota(jnp.int32, (W, WP), 1)
    pmat = (ji // POOL == 2 * (jo % 24) + jo // 24).astype(jnp.float32)
    pooled = [
        jnp.dot(jnp.dot(prt, v, preferred_element_type=jnp.float32), pmat,
                preferred_element_type=jnp.float32)
        for v in vals
    ]

    hs = [pooled[k] - pooled[k + 1] if k < NB - 1 else pooled[k]
          for k in range(NB)]

    ssq = hs[0] * hs[0]
    for k in range(1, NB):
        ssq = ssq + hs[k] * hs[k]
    inv = 1.0 / jnp.maximum(jnp.sqrt(ssq), 1e-12)
    return [h * inv for h in hs]


def _hog_kernel(x_ref, o_ref):
    planes = []
    for c in range(3):
        planes.extend(_channel_hist(x_ref[0, c]))
    hsn = jnp.stack(planes, axis=0)

    nbh = HP // CPB
    nbw = WP // CPB
    qtot = 3 * NB * CPB * CPB

    si = jax.lax.broadcasted_iota(jnp.int32, (54, qtot), 0)
    li = jax.lax.broadcasted_iota(jnp.int32, (54, qtot), 1)
    e0 = (li == 2 * si).astype(jnp.float32)
    e1 = (li == 2 * si + 1).astype(jnp.float32)

    for bh in range(nbh):
        q = hsn[:, 2 * bh:2 * bh + 2, :].reshape(54, WP)
        tq = q.T
        r0 = tq[0:nbw]
        r1 = tq[nbw:2 * nbw]
        tbh = (jnp.dot(r0, e0, preferred_element_type=jnp.float32)
               + jnp.dot(r1, e1, preferred_element_type=jnp.float32))
        o_ref[0, bh * nbw:(bh + 1) * nbw, :] = tbh


def kernel(x, weight_x, weight_y):
    del weight_x, weight_y
    bsz, c = x.shape[0], x.shape[1]
    qq = c * NB * CPB * CPB
    return pl.pallas_call(
        _hog_kernel,
        grid=(bsz,),
        in_specs=[pl.BlockSpec((1, c, H, W), lambda i: (i, 0, 0, 0))],
        out_specs=pl.BlockSpec((1, NBLK, qq), lambda i: (i, 0, 0)),
        out_shape=jax.ShapeDtypeStruct((bsz, NBLK, qq), jnp.float32),
    )(x)
